# baseline (speedup 1.0000x reference)
import functools

import jax
import jax.numpy as jnp
from jax import lax
from jax.experimental import pallas as pl
from jax.experimental.pallas import tpu as pltpu
from jax.experimental.pallas import tpu_sc as plsc

E = 320000
D = 128
W = D // 2
L = 16
NC = 2
NS = 16
NW = NC * NS
PER_TILE = E // NW
C = 400
N_CHUNKS = PER_TILE // C


@functools.partial(
    pl.kernel,
    mesh=plsc.VectorSubcoreMesh(core_axis_name="c", subcore_axis_name="s"),
    out_type=jax.ShapeDtypeStruct((E,), jnp.float32),
    compiler_params=pltpu.CompilerParams(needs_layout_passes=False,
                                         use_tc_tiling_on_sc=False),
    scratch_types=[
        pltpu.VMEM((2, C), jnp.int32),
        pltpu.VMEM((2, C), jnp.int32),
        pltpu.VMEM((2, C, W), jnp.int32),
        pltpu.VMEM((2, C, W), jnp.int32),
        pltpu.VMEM((PER_TILE,), jnp.float32),
        pltpu.SemaphoreType.DMA,
        pltpu.SemaphoreType.DMA,
    ],
)
def _edge_dot(src_hbm, dst_hbm, h_hbm, out_hbm,
              idxu, idxv, ubuf, vbuf, outall, sem0, sem1):
    wid = lax.axis_index("s") * NC + lax.axis_index("c")
    base = wid * PER_TILE
    lanes = lax.iota(jnp.int32, L)
    sems = (sem0, sem1)
    himask = jnp.full((L,), -65536, jnp.int32)

    def lane_shuffle(x, idx):
        return lax.gather(
            x, idx[:, None],
            dimension_numbers=lax.GatherDimensionNumbers(
                offset_dims=(), collapsed_slice_dims=(0,),
                start_index_map=(0,)),
            slice_sizes=(1,),
            mode=lax.GatherScatterMode.PROMISE_IN_BOUNDS)

    def widen(pair_bits):
        lo = lax.bitcast_convert_type(
            lax.shift_left(pair_bits, 16), jnp.float32)
        hi = lax.bitcast_convert_type(
            lax.bitwise_and(pair_bits, himask), jnp.float32)
        return lo, hi

    def start_gather(chunk, buf):
        off = base + chunk * C
        pltpu.sync_copy(src_hbm.at[pl.ds(off, C)], idxu.at[buf])
        pltpu.sync_copy(dst_hbm.at[pl.ds(off, C)], idxv.at[buf])
        cu = pltpu.async_copy(h_hbm.at[idxu.at[buf]], ubuf.at[buf], sems[buf])
        cv = pltpu.async_copy(h_hbm.at[idxv.at[buf]], vbuf.at[buf], sems[buf])
        return cu, cv

    def compute(chunk, buf, cu, cv):
        cu.wait()
        cv.wait()
        urows = ubuf.at[buf]
        vrows = vbuf.at[buf]

        def group_body(g, carry):
            vecs = []
            for j in range(L):
                e = g * L + j
                acc = jnp.zeros((L,), jnp.float32)
                for c in range(W // L):
                    ub = urows[e, pl.ds(c * L, L)]
                    vb = vrows[e, pl.ds(c * L, L)]
                    ulo, uhi = widen(ub)
                    vlo, vhi = widen(vb)
                    acc = acc + ulo * vlo + uhi * vhi
                vecs.append(acc)
            for bit in (8, 4, 2, 1):
                vecs = [v + lane_shuffle(v, lanes ^ bit) for v in vecs]
                vecs = [jnp.where((lanes & bit) == 0, vecs[2 * i], vecs[2 * i + 1])
                        for i in range(len(vecs) // 2)]
            rev = (((lanes & 1) << 3) | ((lanes & 2) << 1)
                   | ((lanes & 4) >> 1) | ((lanes & 8) >> 3))
            tot = lane_shuffle(vecs[0], rev)
            outall[pl.ds(chunk * C + g * L, L)] = tot
            return carry

        lax.fori_loop(0, C // L, group_body, 0)

    cu0, cv0 = start_gather(0, 0)

    def pair_body(k, carry):
        c0 = 2 * k
        cu1, cv1 = start_gather(c0 + 1, 1)
        compute(c0, 0, cu0, cv0)
        start_gather(c0 + 2, 0)
        compute(c0 + 1, 1, cu1, cv1)
        return carry

    lax.fori_loop(0, (N_CHUNKS - 1) // 2, pair_body, 0)
    compute(N_CHUNKS - 1, 0, cu0, cv0)

    pltpu.sync_copy(outall, out_hbm.at[pl.ds(base, PER_TILE)])


def kernel(h, edge_index):
    edge_index = edge_index.astype(jnp.int32)
    src = edge_index[0]
    dst = edge_index[1]
    hb = h.astype(jnp.bfloat16).reshape(h.shape[0], h.shape[1] // 2, 2)
    h32 = lax.bitcast_convert_type(hb, jnp.int32)
    score = _edge_dot(src, dst, h32)
    return score.reshape(E, 1)


# speedup vs baseline: 1.2817x; 1.2817x over previous
import functools

import jax
import jax.numpy as jnp
from jax import lax
from jax.experimental import pallas as pl
from jax.experimental.pallas import tpu as pltpu
from jax.experimental.pallas import tpu_sc as plsc

E = 320000
D = 128
W = D // 2
L = 16
NC = 2
NS = 16
NW = NC * NS
PER_TILE = E // NW
C = 400
N_CHUNKS = PER_TILE // C


@functools.partial(
    pl.kernel,
    mesh=plsc.VectorSubcoreMesh(core_axis_name="c", subcore_axis_name="s"),
    out_type=jax.ShapeDtypeStruct((E,), jnp.float32),
    compiler_params=pltpu.CompilerParams(needs_layout_passes=False,
                                         use_tc_tiling_on_sc=False),
    scratch_types=[
        pltpu.VMEM((2, C), jnp.int32),
        pltpu.VMEM((2, C), jnp.int32),
        pltpu.VMEM((2, C, W), jnp.int32),
        pltpu.VMEM((2, C, W), jnp.int32),
        pltpu.VMEM((PER_TILE,), jnp.float32),
        pltpu.SemaphoreType.DMA,
        pltpu.SemaphoreType.DMA,
    ],
)
def _edge_dot(src_hbm, dst_hbm, h_hbm, out_hbm,
              idxu, idxv, ubuf, vbuf, outall, sem0, sem1):
    wid = lax.axis_index("s") * NC + lax.axis_index("c")
    base = wid * PER_TILE
    lanes = lax.iota(jnp.int32, L)
    sems = (sem0, sem1)
    himask = jnp.full((L,), -65536, jnp.int32)

    def lane_shuffle(x, idx):
        return lax.gather(
            x, idx[:, None],
            dimension_numbers=lax.GatherDimensionNumbers(
                offset_dims=(), collapsed_slice_dims=(0,),
                start_index_map=(0,)),
            slice_sizes=(1,),
            mode=lax.GatherScatterMode.PROMISE_IN_BOUNDS)

    def widen(pair_bits):
        lo = lax.bitcast_convert_type(
            lax.shift_left(pair_bits, 16), jnp.float32)
        hi = lax.bitcast_convert_type(
            lax.bitwise_and(pair_bits, himask), jnp.float32)
        return lo, hi

    def start_gather(chunk, buf):
        off = base + chunk * C
        pltpu.sync_copy(src_hbm.at[pl.ds(off, C)], idxu.at[buf])
        pltpu.sync_copy(dst_hbm.at[pl.ds(off, C)], idxv.at[buf])
        cu = pltpu.async_copy(h_hbm.at[idxu.at[buf]], ubuf.at[buf], sems[buf])
        cv = pltpu.async_copy(h_hbm.at[idxv.at[buf]], vbuf.at[buf], sems[buf])
        return cu, cv

    def compute(chunk, buf, cu, cv):
        cu.wait()
        cv.wait()
        urows = ubuf.at[buf]
        vrows = vbuf.at[buf]

        def group_body(g, carry):
            tot = jnp.zeros((L,), jnp.float32)
            for j in range(L):
                e = g * L + j
                acc_lo = jnp.zeros((L,), jnp.float32)
                acc_hi = jnp.zeros((L,), jnp.float32)
                for c in range(W // L):
                    ub = urows[e, pl.ds(c * L, L)]
                    vb = vrows[e, pl.ds(c * L, L)]
                    prod = (plsc.bitcast(ub, jnp.bfloat16)
                            * plsc.bitcast(vb, jnp.bfloat16))
                    plo, phi = widen(plsc.bitcast(prod, jnp.int32))
                    acc_lo = acc_lo + plo
                    acc_hi = acc_hi + phi
                acc = acc_lo + acc_hi
                for dist in (8, 4, 2, 1):
                    acc = acc + lane_shuffle(acc, lanes ^ dist)
                tot = jnp.where(lanes == j, acc, tot)
            outall[pl.ds(chunk * C + g * L, L)] = tot
            return carry

        lax.fori_loop(0, C // L, group_body, 0)

    cu0, cv0 = start_gather(0, 0)

    def pair_body(k, carry):
        c0 = 2 * k
        cu1, cv1 = start_gather(c0 + 1, 1)
        compute(c0, 0, cu0, cv0)
        start_gather(c0 + 2, 0)
        compute(c0 + 1, 1, cu1, cv1)
        return carry

    lax.fori_loop(0, (N_CHUNKS - 1) // 2, pair_body, 0)
    compute(N_CHUNKS - 1, 0, cu0, cv0)

    pltpu.sync_copy(outall, out_hbm.at[pl.ds(base, PER_TILE)])


def kernel(h, edge_index):
    edge_index = edge_index.astype(jnp.int32)
    src = edge_index[0]
    dst = edge_index[1]
    hb = h.astype(jnp.bfloat16).reshape(h.shape[0], h.shape[1] // 2, 2)
    h32 = lax.bitcast_convert_type(hb, jnp.int32)
    score = _edge_dot(src, dst, h32)
    return score.reshape(E, 1)


# speedup vs baseline: 1.4416x; 1.1248x over previous
import functools

import jax
import jax.numpy as jnp
from jax import lax
from jax.experimental import pallas as pl
from jax.experimental.pallas import tpu as pltpu
from jax.experimental.pallas import tpu_sc as plsc

E = 320000
D = 128
W = D // 2
L = 16
NC = 2
NS = 16
NW = NC * NS
PER_TILE = E // NW
C = 400
N_CHUNKS = PER_TILE // C


@functools.partial(
    pl.kernel,
    mesh=plsc.VectorSubcoreMesh(core_axis_name="c", subcore_axis_name="s"),
    out_type=jax.ShapeDtypeStruct((E,), jnp.float32),
    compiler_params=pltpu.CompilerParams(needs_layout_passes=False,
                                         use_tc_tiling_on_sc=False),
    scratch_types=[
        pltpu.VMEM((2, C), jnp.int32),
        pltpu.VMEM((2, C), jnp.int32),
        pltpu.VMEM((2, C, W), jnp.int32),
        pltpu.VMEM((2, C, W), jnp.int32),
        pltpu.VMEM((PER_TILE,), jnp.float32),
        pltpu.SemaphoreType.DMA,
        pltpu.SemaphoreType.DMA,
        pltpu.SemaphoreType.DMA,
        pltpu.SemaphoreType.DMA,
    ],
)
def _edge_dot(src_hbm, dst_hbm, h_hbm, out_hbm,
              idxu, idxv, ubuf, vbuf, outall, sem0, sem1, isem0, isem1):
    wid = lax.axis_index("s") * NC + lax.axis_index("c")
    base = wid * PER_TILE
    lanes = lax.iota(jnp.int32, L)
    sems = (sem0, sem1)
    himask = jnp.full((L,), -65536, jnp.int32)

    def lane_shuffle(x, idx):
        return lax.gather(
            x, idx[:, None],
            dimension_numbers=lax.GatherDimensionNumbers(
                offset_dims=(), collapsed_slice_dims=(0,),
                start_index_map=(0,)),
            slice_sizes=(1,),
            mode=lax.GatherScatterMode.PROMISE_IN_BOUNDS)

    def widen(pair_bits):
        lo = lax.bitcast_convert_type(
            lax.shift_left(pair_bits, 16), jnp.float32)
        hi = lax.bitcast_convert_type(
            lax.bitwise_and(pair_bits, himask), jnp.float32)
        return lo, hi

    isems = (isem0, isem1)

    def start_idx(chunk, buf):
        off = base + chunk * C
        ci = pltpu.async_copy(src_hbm.at[pl.ds(off, C)], idxu.at[buf],
                              isems[buf])
        cj = pltpu.async_copy(dst_hbm.at[pl.ds(off, C)], idxv.at[buf],
                              isems[buf])
        return ci, cj

    def start_gather(buf, ci, cj):
        ci.wait()
        cj.wait()
        cu = pltpu.async_copy(h_hbm.at[idxu.at[buf]], ubuf.at[buf], sems[buf])
        cv = pltpu.async_copy(h_hbm.at[idxv.at[buf]], vbuf.at[buf], sems[buf])
        return cu, cv

    def compute(chunk, buf):
        urows = ubuf.at[buf]
        vrows = vbuf.at[buf]

        def group_body(g, carry):
            tot = jnp.zeros((L,), jnp.float32)
            for j in range(L):
                e = g * L + j
                acc_lo = jnp.zeros((L,), jnp.float32)
                acc_hi = jnp.zeros((L,), jnp.float32)
                for c in range(W // L):
                    ub = urows[e, pl.ds(c * L, L)]
                    vb = vrows[e, pl.ds(c * L, L)]
                    prod = (plsc.bitcast(ub, jnp.bfloat16)
                            * plsc.bitcast(vb, jnp.bfloat16))
                    plo, phi = widen(plsc.bitcast(prod, jnp.int32))
                    acc_lo = acc_lo + plo
                    acc_hi = acc_hi + phi
                acc = acc_lo + acc_hi
                for dist in (8, 4, 2, 1):
                    acc = acc + lane_shuffle(acc, lanes ^ dist)
                tot = jnp.where(lanes == j, acc, tot)
            outall[pl.ds(chunk * C + g * L, L)] = tot
            return carry

        lax.fori_loop(0, C // L, group_body, 0)

    i0 = start_idx(0, 0)
    i1 = start_idx(1, 1)
    cu0, cv0 = start_gather(0, *i0)
    cu1, cv1 = start_gather(1, *i1)

    def pair_body(k, carry):
        a = 2 * k
        cu0.wait()
        cv0.wait()
        ia = start_idx(a + 2, 0)
        compute(a, 0)
        start_gather(0, *ia)
        cu1.wait()
        cv1.wait()
        ib = start_idx(a + 3, 1)
        compute(a + 1, 1)
        start_gather(1, *ib)
        return carry

    lax.fori_loop(0, (N_CHUNKS - 3) // 2, pair_body, 0)
    cu0.wait()
    cv0.wait()
    ia = start_idx(N_CHUNKS - 1, 0)
    compute(N_CHUNKS - 3, 0)
    start_gather(0, *ia)
    cu1.wait()
    cv1.wait()
    compute(N_CHUNKS - 2, 1)
    cu0.wait()
    cv0.wait()
    compute(N_CHUNKS - 1, 0)

    pltpu.sync_copy(outall, out_hbm.at[pl.ds(base, PER_TILE)])


def kernel(h, edge_index):
    edge_index = edge_index.astype(jnp.int32)
    src = edge_index[0]
    dst = edge_index[1]
    hb = h.astype(jnp.bfloat16).reshape(h.shape[0], h.shape[1] // 2, 2)
    h32 = lax.bitcast_convert_type(hb, jnp.int32)
    score = _edge_dot(src, dst, h32)
    return score.reshape(E, 1)


# speedup vs baseline: 1.5296x; 1.0611x over previous
import functools

import jax
import jax.numpy as jnp
from jax import lax
from jax.experimental import pallas as pl
from jax.experimental.pallas import tpu as pltpu
from jax.experimental.pallas import tpu_sc as plsc

E = 320000
D = 128
W = D // 2
L = 16
NC = 2
NS = 16
NW = NC * NS
PER_TILE = E // NW
C = 400
N_CHUNKS = PER_TILE // C


@functools.partial(
    pl.kernel,
    mesh=plsc.VectorSubcoreMesh(core_axis_name="c", subcore_axis_name="s"),
    out_type=jax.ShapeDtypeStruct((E,), jnp.float32),
    compiler_params=pltpu.CompilerParams(needs_layout_passes=False,
                                         use_tc_tiling_on_sc=False),
    scratch_types=[
        pltpu.VMEM((2, 2 * C), jnp.int32),
        pltpu.VMEM((2, 2 * C, W), jnp.int32),
        pltpu.VMEM((PER_TILE,), jnp.float32),
        pltpu.SemaphoreType.DMA,
        pltpu.SemaphoreType.DMA,
        pltpu.SemaphoreType.DMA,
        pltpu.SemaphoreType.DMA,
    ],
)
def _edge_dot(edge_hbm, h_hbm, out_hbm,
              idx, rbuf, outall, sem0, sem1, isem0, isem1):
    wid = lax.axis_index("s") * NC + lax.axis_index("c")
    base = wid * PER_TILE
    lanes = lax.iota(jnp.int32, L)
    sems = (sem0, sem1)
    isems = (isem0, isem1)
    himask = jnp.full((L,), -65536, jnp.int32)

    def lane_shuffle(x, i):
        return lax.gather(
            x, i[:, None],
            dimension_numbers=lax.GatherDimensionNumbers(
                offset_dims=(), collapsed_slice_dims=(0,),
                start_index_map=(0,)),
            slice_sizes=(1,),
            mode=lax.GatherScatterMode.PROMISE_IN_BOUNDS)

    def widen(pair_bits):
        lo = lax.bitcast_convert_type(
            lax.shift_left(pair_bits, 16), jnp.float32)
        hi = lax.bitcast_convert_type(
            lax.bitwise_and(pair_bits, himask), jnp.float32)
        return lo, hi

    def start_idx(chunk, buf):
        off = base + chunk * C
        ci = pltpu.async_copy(edge_hbm.at[0, pl.ds(off, C)],
                              idx.at[buf, pl.ds(0, C)], isems[buf])
        cj = pltpu.async_copy(edge_hbm.at[1, pl.ds(off, C)],
                              idx.at[buf, pl.ds(C, C)], isems[buf])
        return ci, cj

    def start_gather(buf, ci, cj):
        ci.wait()
        cj.wait()
        return pltpu.async_copy(h_hbm.at[idx.at[buf]], rbuf.at[buf],
                                sems[buf])

    def compute(chunk, buf):
        urows = rbuf.at[buf, pl.ds(0, C)]
        vrows = rbuf.at[buf, pl.ds(C, C)]

        def group_body(g, carry):
            tot = jnp.zeros((L,), jnp.float32)
            for j in range(L):
                e = g * L + j
                acc_lo = jnp.zeros((L,), jnp.float32)
                acc_hi = jnp.zeros((L,), jnp.float32)
                for c in range(W // L):
                    ub = urows[e, pl.ds(c * L, L)]
                    vb = vrows[e, pl.ds(c * L, L)]
                    prod = (plsc.bitcast(ub, jnp.bfloat16)
                            * plsc.bitcast(vb, jnp.bfloat16))
                    plo, phi = widen(plsc.bitcast(prod, jnp.int32))
                    acc_lo = acc_lo + plo
                    acc_hi = acc_hi + phi
                acc = acc_lo + acc_hi
                for dist in (8, 4, 2, 1):
                    acc = acc + lane_shuffle(acc, lanes ^ dist)
                tot = jnp.where(lanes == j, acc, tot)
            outall[pl.ds(chunk * C + g * L, L)] = tot
            return carry

        lax.fori_loop(0, C // L, group_body, 0)

    i0 = start_idx(0, 0)
    i1 = start_idx(1, 1)
    g0 = start_gather(0, *i0)
    g1 = start_gather(1, *i1)

    def pair_body(k, carry):
        a = 2 * k
        g0.wait()
        ia = start_idx(a + 2, 0)
        compute(a, 0)
        start_gather(0, *ia)
        g1.wait()
        ib = start_idx(a + 3, 1)
        compute(a + 1, 1)
        start_gather(1, *ib)
        return carry

    lax.fori_loop(0, (N_CHUNKS - 3) // 2, pair_body, 0)
    g0.wait()
    ia = start_idx(N_CHUNKS - 1, 0)
    compute(N_CHUNKS - 3, 0)
    start_gather(0, *ia)
    g1.wait()
    compute(N_CHUNKS - 2, 1)
    g0.wait()
    compute(N_CHUNKS - 1, 0)

    pltpu.sync_copy(outall, out_hbm.at[pl.ds(base, PER_TILE)])


def kernel(h, edge_index):
    edge_index = edge_index.astype(jnp.int32)
    hb = h.astype(jnp.bfloat16).reshape(h.shape[0], h.shape[1] // 2, 2)
    h32 = lax.bitcast_convert_type(hb, jnp.int32)
    score = _edge_dot(edge_index, h32)
    return score.reshape(E, 1)


# speedup vs baseline: 1.5534x; 1.0155x over previous
import functools

import jax
import jax.numpy as jnp
from jax import lax
from jax.experimental import pallas as pl
from jax.experimental.pallas import tpu as pltpu
from jax.experimental.pallas import tpu_sc as plsc

E = 320000
D = 128
W = D // 2
L = 16
NC = 2
NS = 16
NW = NC * NS
PER_TILE = E // NW
C = 400
N_CHUNKS = PER_TILE // C


@functools.partial(
    pl.kernel,
    mesh=plsc.VectorSubcoreMesh(core_axis_name="c", subcore_axis_name="s"),
    out_type=jax.ShapeDtypeStruct((E,), jnp.float32),
    compiler_params=pltpu.CompilerParams(needs_layout_passes=False,
                                         use_tc_tiling_on_sc=False),
    scratch_types=[
        pltpu.VMEM((2, 2 * C), jnp.int32),
        pltpu.VMEM((2, 2 * C, W), jnp.int32),
        pltpu.VMEM((PER_TILE,), jnp.float32),
        pltpu.SemaphoreType.DMA,
        pltpu.SemaphoreType.DMA,
        pltpu.SemaphoreType.DMA,
        pltpu.SemaphoreType.DMA,
    ],
)
def _edge_dot(edge_hbm, h_hbm, out_hbm,
              idx, rbuf, outall, sem0, sem1, isem0, isem1):
    wid = lax.axis_index("s") * NC + lax.axis_index("c")
    base = wid * PER_TILE
    lanes = lax.iota(jnp.int32, L)
    sems = (sem0, sem1)
    isems = (isem0, isem1)
    def lane_shuffle(x, i):
        return lax.gather(
            x, i[:, None],
            dimension_numbers=lax.GatherDimensionNumbers(
                offset_dims=(), collapsed_slice_dims=(0,),
                start_index_map=(0,)),
            slice_sizes=(1,),
            mode=lax.GatherScatterMode.PROMISE_IN_BOUNDS)

    def start_idx(chunk, buf):
        off = base + chunk * C
        ci = pltpu.async_copy(edge_hbm.at[0, pl.ds(off, C)],
                              idx.at[buf, pl.ds(0, C)], isems[buf])
        cj = pltpu.async_copy(edge_hbm.at[1, pl.ds(off, C)],
                              idx.at[buf, pl.ds(C, C)], isems[buf])
        return ci, cj

    def start_gather(buf, ci, cj):
        ci.wait()
        cj.wait()
        return pltpu.async_copy(h_hbm.at[idx.at[buf]], rbuf.at[buf],
                                sems[buf])

    lanes2 = lanes >> 1

    def compute(chunk, buf):
        urows = rbuf.at[buf, pl.ds(0, C)]
        vrows = rbuf.at[buf, pl.ds(C, C)]

        def edge_partials(e):
            acc_lo = jnp.zeros((L,), jnp.float32)
            acc_hi = jnp.zeros((L,), jnp.float32)
            for c in range(W // L):
                ub = urows[e, pl.ds(c * L, L)]
                vb = vrows[e, pl.ds(c * L, L)]
                prod = (plsc.bitcast(ub, jnp.bfloat16)
                        * plsc.bitcast(vb, jnp.bfloat16))
                p32 = plsc.bitcast(prod, jnp.int32)
                acc_lo = acc_lo + lax.bitcast_convert_type(
                    lax.shift_left(p32, 16), jnp.float32)
                acc_hi = acc_hi + lax.bitcast_convert_type(p32, jnp.float32)
            return acc_lo + acc_hi

        def group_body(g, carry):
            tot = jnp.zeros((L,), jnp.float32)
            for i in range(L // 2):
                fa = edge_partials(g * L + 2 * i)
                fb = edge_partials(g * L + 2 * i + 1)
                fa = fa + lane_shuffle(fa, lanes ^ 1)
                fb = fb + lane_shuffle(fb, lanes ^ 1)
                m = jnp.where((lanes & 1) == 0, fa, fb)
                for dist in (8, 4, 2):
                    m = m + lane_shuffle(m, lanes ^ dist)
                tot = jnp.where(lanes2 == i, m, tot)
            outall[pl.ds(chunk * C + g * L, L)] = tot
            return carry

        lax.fori_loop(0, C // L, group_body, 0)

    i0 = start_idx(0, 0)
    i1 = start_idx(1, 1)
    g0 = start_gather(0, *i0)
    g1 = start_gather(1, *i1)

    def pair_body(k, carry):
        a = 2 * k
        g0.wait()
        ia = start_idx(a + 2, 0)
        compute(a, 0)
        start_gather(0, *ia)
        g1.wait()
        ib = start_idx(a + 3, 1)
        compute(a + 1, 1)
        start_gather(1, *ib)
        return carry

    lax.fori_loop(0, (N_CHUNKS - 3) // 2, pair_body, 0)
    g0.wait()
    ia = start_idx(N_CHUNKS - 1, 0)
    compute(N_CHUNKS - 3, 0)
    start_gather(0, *ia)
    g1.wait()
    compute(N_CHUNKS - 2, 1)
    g0.wait()
    compute(N_CHUNKS - 1, 0)

    pltpu.sync_copy(outall, out_hbm.at[pl.ds(base, PER_TILE)])


def kernel(h, edge_index):
    edge_index = edge_index.astype(jnp.int32)
    hb = h.astype(jnp.bfloat16).reshape(h.shape[0], h.shape[1] // 2, 2)
    h32 = lax.bitcast_convert_type(hb, jnp.int32)
    score = _edge_dot(edge_index, h32)
    return score.reshape(E, 1)


# speedup vs baseline: 1.5984x; 1.0290x over previous
import functools

import jax
import jax.numpy as jnp
from jax import lax
from jax.experimental import pallas as pl
from jax.experimental.pallas import tpu as pltpu
from jax.experimental.pallas import tpu_sc as plsc

E = 320000
D = 128
W = D // 2
L = 16
NC = 2
NS = 16
NW = NC * NS
PER_TILE = E // NW
C = 400
N_CHUNKS = PER_TILE // C


@functools.partial(
    pl.kernel,
    mesh=plsc.VectorSubcoreMesh(core_axis_name="c", subcore_axis_name="s"),
    out_type=jax.ShapeDtypeStruct((E,), jnp.float32),
    compiler_params=pltpu.CompilerParams(needs_layout_passes=False,
                                         use_tc_tiling_on_sc=False),
    scratch_types=[
        pltpu.VMEM((2, 2 * C), jnp.int32),
        pltpu.VMEM((2, 2 * C, W), jnp.int32),
        pltpu.VMEM((PER_TILE,), jnp.float32),
        pltpu.SemaphoreType.DMA,
        pltpu.SemaphoreType.DMA,
        pltpu.SemaphoreType.DMA,
        pltpu.SemaphoreType.DMA,
    ],
)
def _edge_dot(edge_hbm, h_hbm, out_hbm,
              idx, rbuf, outall, sem0, sem1, isem0, isem1):
    wid = lax.axis_index("s") * NC + lax.axis_index("c")
    base = wid * PER_TILE
    lanes = lax.iota(jnp.int32, L)
    sems = (sem0, sem1)
    isems = (isem0, isem1)
    def lane_shuffle(x, i):
        return lax.gather(
            x, i[:, None],
            dimension_numbers=lax.GatherDimensionNumbers(
                offset_dims=(), collapsed_slice_dims=(0,),
                start_index_map=(0,)),
            slice_sizes=(1,),
            mode=lax.GatherScatterMode.PROMISE_IN_BOUNDS)

    def start_idx(chunk, buf):
        off = base + chunk * C
        ci = pltpu.async_copy(edge_hbm.at[0, pl.ds(off, C)],
                              idx.at[buf, pl.ds(0, C)], isems[buf])
        cj = pltpu.async_copy(edge_hbm.at[1, pl.ds(off, C)],
                              idx.at[buf, pl.ds(C, C)], isems[buf])
        return ci, cj

    def start_gather(buf, ci, cj):
        ci.wait()
        cj.wait()
        return pltpu.async_copy(h_hbm.at[idx.at[buf]], rbuf.at[buf],
                                sems[buf])

    lanes2 = lanes >> 1

    def compute(chunk, buf):
        urows = rbuf.at[buf, pl.ds(0, C)]
        vrows = rbuf.at[buf, pl.ds(C, C)]

        def edge_partials(e):
            acc_lo = jnp.zeros((L,), jnp.float32)
            acc_hi = jnp.zeros((L,), jnp.float32)
            for c in range(W // L):
                ub = urows[e, pl.ds(c * L, L)]
                vb = vrows[e, pl.ds(c * L, L)]
                prod = (plsc.bitcast(ub, jnp.bfloat16)
                        * plsc.bitcast(vb, jnp.bfloat16))
                p32 = plsc.bitcast(prod, jnp.int32)
                acc_lo = acc_lo + lax.bitcast_convert_type(
                    lax.shift_left(p32, 16), jnp.float32)
                acc_hi = acc_hi + lax.bitcast_convert_type(p32, jnp.float32)
            return acc_lo + acc_hi

        def one_group(g):
            tot = jnp.zeros((L,), jnp.float32)
            for i in range(L // 2):
                fa = edge_partials(g * L + 2 * i)
                fb = edge_partials(g * L + 2 * i + 1)
                fa = fa + lane_shuffle(fa, lanes ^ 1)
                fb = fb + lane_shuffle(fb, lanes ^ 1)
                m = jnp.where((lanes & 1) == 0, fa, fb)
                for dist in (8, 4, 2):
                    m = m + lane_shuffle(m, lanes ^ dist)
                tot = jnp.where(lanes2 == i, m, tot)
            outall[pl.ds(chunk * C + g * L, L)] = tot

        def group_body(k, carry):
            one_group(2 * k)
            one_group(2 * k + 1)
            return carry

        lax.fori_loop(0, C // L // 2, group_body, 0)

    i0 = start_idx(0, 0)
    i1 = start_idx(1, 1)
    g0 = start_gather(0, *i0)
    g1 = start_gather(1, *i1)

    def pair_body(k, carry):
        a = 2 * k
        g0.wait()
        ia = start_idx(a + 2, 0)
        compute(a, 0)
        start_gather(0, *ia)
        g1.wait()
        ib = start_idx(a + 3, 1)
        compute(a + 1, 1)
        start_gather(1, *ib)
        return carry

    lax.fori_loop(0, (N_CHUNKS - 3) // 2, pair_body, 0)
    g0.wait()
    ia = start_idx(N_CHUNKS - 1, 0)
    compute(N_CHUNKS - 3, 0)
    start_gather(0, *ia)
    g1.wait()
    compute(N_CHUNKS - 2, 1)
    g0.wait()
    compute(N_CHUNKS - 1, 0)

    pltpu.sync_copy(outall, out_hbm.at[pl.ds(base, PER_TILE)])


def kernel(h, edge_index):
    edge_index = edge_index.astype(jnp.int32)
    hb = h.astype(jnp.bfloat16).reshape(h.shape[0], h.shape[1] // 2, 2)
    h32 = lax.bitcast_convert_type(hb, jnp.int32)
    score = _edge_dot(edge_index, h32)
    return score.reshape(E, 1)
